# Initial kernel scaffold; baseline (speedup 1.0000x reference)
import jax
import jax.numpy as jnp
from jax.experimental import pallas as pl


def kernel(x, edge_index, batch, W1, b1, W2, b2, fc1_W, fc1_b, fc2_W, fc2_b):
    raise NotImplementedError("write your pallas kernel here")


# speedup vs baseline: 12.3196x; 12.3196x over previous
import functools

import jax
import jax.numpy as jnp
from jax import lax
from jax.experimental import pallas as pl
from jax.experimental.pallas import tpu as pltpu
from jax.experimental.pallas import tpu_sc as plsc

_N = 10000
_E = 320000
_D = 128
_G = 16
_OUT = 10
_NC = 2
_NS = 16
_K = 80
_EPT = _E // (_NC * _NS)
_NCHUNK = _EPT // _K
_NPAD = 10240
_RPT = _NPAD // _NS
_R = 1000
_NB = _N // _R


def _sc_mesh():
    return plsc.VectorSubcoreMesh(
        core_axis_name="c", subcore_axis_name="s",
        num_cores=_NC, num_subcores=_NS)


def _sc_degree(dst):
    ones_rows = jnp.ones((_K, _D), jnp.float32)
    zeros_tile = jnp.zeros((_RPT, _D), jnp.float32)

    @functools.partial(
        pl.kernel,
        out_type=jax.ShapeDtypeStruct((_NC, _NPAD, _D), jnp.float32),
        mesh=_sc_mesh(),
        scratch_types=[
            pltpu.VMEM((_K,), jnp.int32),
            pltpu.VMEM((_K, _D), jnp.float32),
            pltpu.VMEM_SHARED((_NPAD, _D), jnp.float32),
        ],
    )
    def deg_kernel(dst_hbm, ones_hbm, zeros_hbm, out_hbm, idx_v, ones_v, deg_sh):
        c = lax.axis_index("c")
        s = lax.axis_index("s")
        pltpu.sync_copy(ones_hbm, ones_v)
        pltpu.sync_copy(zeros_hbm, deg_sh.at[pl.ds(s * _RPT, _RPT)])
        plsc.subcore_barrier()
        ebase = (c * _NS + s) * _EPT

        def body(j, carry):
            pltpu.sync_copy(dst_hbm.at[pl.ds(ebase + j * _K, _K)], idx_v)
            pltpu.sync_copy(ones_v, deg_sh.at[idx_v], add=True)
            return carry

        lax.fori_loop(0, _NCHUNK, body, 0)
        plsc.subcore_barrier()
        pltpu.sync_copy(deg_sh.at[pl.ds(s * _RPT, _RPT)],
                        out_hbm.at[c, pl.ds(s * _RPT, _RPT)])

    return deg_kernel(dst, ones_rows, zeros_tile)


def _sc_edge_scatter(src, dst, g):
    zeros_tile = jnp.zeros((_RPT, _D), jnp.float32)

    @functools.partial(
        pl.kernel,
        out_type=jax.ShapeDtypeStruct((_NC, _NPAD, _D), jnp.float32),
        mesh=_sc_mesh(),
        scratch_types=[
            pltpu.VMEM((_K,), jnp.int32),
            pltpu.VMEM((_K,), jnp.int32),
            pltpu.VMEM((_K, _D), jnp.float32),
            pltpu.VMEM_SHARED((_NPAD, _D), jnp.float32),
            pltpu.SemaphoreType.DMA,
        ],
    )
    def edge_kernel(src_hbm, dst_hbm, g_hbm, zeros_hbm, out_hbm,
                    idx_s, idx_d, rows_v, agg_sh, sem):
        c = lax.axis_index("c")
        s = lax.axis_index("s")
        pltpu.sync_copy(zeros_hbm, agg_sh.at[pl.ds(s * _RPT, _RPT)])
        plsc.subcore_barrier()
        ebase = (c * _NS + s) * _EPT

        def body(j, carry):
            pltpu.sync_copy(src_hbm.at[pl.ds(ebase + j * _K, _K)], idx_s)
            pltpu.sync_copy(dst_hbm.at[pl.ds(ebase + j * _K, _K)], idx_d)
            pltpu.async_copy(g_hbm.at[idx_s], rows_v, sem).wait()
            pltpu.sync_copy(rows_v, agg_sh.at[idx_d], add=True)
            return carry

        lax.fori_loop(0, _NCHUNK, body, 0)
        plsc.subcore_barrier()
        pltpu.sync_copy(agg_sh.at[pl.ds(s * _RPT, _RPT)],
                        out_hbm.at[c, pl.ds(s * _RPT, _RPT)])

    return edge_kernel(src, dst, g, zeros_tile)


def _dinv_block(d_ref):
    deg = d_ref[0, :, 0:1] + d_ref[1, :, 0:1] + 1.0
    return lax.rsqrt(deg)


def _tc_first(x, W1, degp):
    def body(x_ref, w_ref, d_ref, o_ref):
        dinv = _dinv_block(d_ref)
        o_ref[...] = jnp.dot(x_ref[...], w_ref[...],
                             preferred_element_type=jnp.float32) * dinv

    return pl.pallas_call(
        body,
        grid=(_NB,),
        in_specs=[
            pl.BlockSpec((_R, _D), lambda i: (i, 0)),
            pl.BlockSpec((_D, _D), lambda i: (0, 0)),
            pl.BlockSpec((_NC, _R, _D), lambda i: (0, i, 0)),
        ],
        out_specs=pl.BlockSpec((_R, _D), lambda i: (i, 0)),
        out_shape=jax.ShapeDtypeStruct((_N, _D), jnp.float32),
    )(x, W1, degp)


def _tc_mid(acc, g, degp, b, W2):
    def body(a_ref, g_ref, d_ref, b_ref, w_ref, o_ref):
        dinv = _dinv_block(d_ref)
        agg = dinv * (a_ref[0] + a_ref[1] + g_ref[...]) + b_ref[...]
        x2 = jnp.maximum(agg, 0.0)
        o_ref[...] = jnp.dot(x2, w_ref[...],
                             preferred_element_type=jnp.float32) * dinv

    return pl.pallas_call(
        body,
        grid=(_NB,),
        in_specs=[
            pl.BlockSpec((_NC, _R, _D), lambda i: (0, i, 0)),
            pl.BlockSpec((_R, _D), lambda i: (i, 0)),
            pl.BlockSpec((_NC, _R, _D), lambda i: (0, i, 0)),
            pl.BlockSpec((1, _D), lambda i: (0, 0)),
            pl.BlockSpec((_D, _D), lambda i: (0, 0)),
        ],
        out_specs=pl.BlockSpec((_R, _D), lambda i: (i, 0)),
        out_shape=jax.ShapeDtypeStruct((_N, _D), jnp.float32),
    )(acc, g, degp, b, W2)


def _tc_final(acc, g, degp, b, batch3, fc1_W, fc1_b, fc2_Wp, fc2_bp):
    def body(a_ref, g_ref, d_ref, b_ref, bt_ref, w1_ref, b1_ref, w2_ref, b2_ref,
             o_ref, sum_sc, cnt_sc):
        i = pl.program_id(0)

        @pl.when(i == 0)
        def _init():
            sum_sc[...] = jnp.zeros_like(sum_sc)
            cnt_sc[...] = jnp.zeros_like(cnt_sc)

        dinv = _dinv_block(d_ref)
        agg = dinv * (a_ref[0] + a_ref[1] + g_ref[...]) + b_ref[...]
        x3 = jnp.maximum(agg, 0.0)
        bt = bt_ref[0, 0, :]
        onehot = (bt[None, :] ==
                  lax.broadcasted_iota(jnp.int32, (_G, _R), 0)).astype(jnp.float32)
        sum_sc[...] += jnp.dot(onehot, x3, preferred_element_type=jnp.float32)
        cnt_sc[...] += jnp.sum(onehot, axis=1, keepdims=True)

        @pl.when(i == _NB - 1)
        def _finish():
            pooled = sum_sc[...] / jnp.maximum(cnt_sc[...], 1.0)
            h = jnp.maximum(
                jnp.dot(pooled, w1_ref[...],
                        preferred_element_type=jnp.float32) + b1_ref[...], 0.0)
            logits = jnp.dot(h, w2_ref[...],
                             preferred_element_type=jnp.float32) + b2_ref[...]
            col = lax.broadcasted_iota(jnp.int32, (_G, _D), 1)
            logits = jnp.where(col < _OUT, logits, -1e30)
            m = jnp.max(logits, axis=1, keepdims=True)
            e = jnp.exp(logits - m)
            e = jnp.where(col < _OUT, e, 0.0)
            o_ref[...] = e / jnp.sum(e, axis=1, keepdims=True)

    return pl.pallas_call(
        body,
        grid=(_NB,),
        in_specs=[
            pl.BlockSpec((_NC, _R, _D), lambda i: (0, i, 0)),
            pl.BlockSpec((_R, _D), lambda i: (i, 0)),
            pl.BlockSpec((_NC, _R, _D), lambda i: (0, i, 0)),
            pl.BlockSpec((1, _D), lambda i: (0, 0)),
            pl.BlockSpec((1, 1, _R), lambda i: (i, 0, 0)),
            pl.BlockSpec((_D, _D), lambda i: (0, 0)),
            pl.BlockSpec((1, _D), lambda i: (0, 0)),
            pl.BlockSpec((_D, _D), lambda i: (0, 0)),
            pl.BlockSpec((1, _D), lambda i: (0, 0)),
        ],
        out_specs=pl.BlockSpec((_G, _D), lambda i: (0, 0)),
        out_shape=jax.ShapeDtypeStruct((_G, _D), jnp.float32),
        scratch_shapes=[
            pltpu.VMEM((_G, _D), jnp.float32),
            pltpu.VMEM((_G, 1), jnp.float32),
        ],
    )(acc, g, degp, b, batch3, fc1_W, fc1_b, fc2_Wp, fc2_bp)


def kernel(x, edge_index, batch, W1, b1, W2, b2, fc1_W, fc1_b, fc2_W, fc2_b):
    src = edge_index[0].astype(jnp.int32)
    dst = edge_index[1].astype(jnp.int32)
    batch3 = batch.astype(jnp.int32).reshape(_NB, 1, _R)

    degp = _sc_degree(dst)
    g1 = _tc_first(x, W1, degp)
    acc1 = _sc_edge_scatter(src, dst, g1)
    g2 = _tc_mid(acc1, g1, degp, b1.reshape(1, _D), W2)
    acc2 = _sc_edge_scatter(src, dst, g2)

    fc2_Wp = jnp.zeros((_D, _D), jnp.float32).at[:, :_OUT].set(fc2_W)
    fc2_bp = jnp.zeros((1, _D), jnp.float32).at[0, :_OUT].set(fc2_b)
    out = _tc_final(acc2, g2, degp, b2.reshape(1, _D), batch3,
                    fc1_W, fc1_b.reshape(1, _D), fc2_Wp, fc2_bp)
    return out[:, :_OUT]


# speedup vs baseline: 25.5111x; 2.0708x over previous
import functools

import jax
import jax.numpy as jnp
from jax import lax
from jax.experimental import pallas as pl
from jax.experimental.pallas import tpu as pltpu
from jax.experimental.pallas import tpu_sc as plsc

_N = 10000
_E = 320000
_D = 128
_G = 16
_OUT = 10
_NC = 2
_NS = 16
_K = 128
_NCHUNK = 80
_EPT = _NCHUNK * _K
_EPAD = _NC * _NS * _EPT
_NPAD = 10240
_RPT = _NPAD // _NS
_R = 1000
_NB = _N // _R


def _sc_mesh():
    return plsc.VectorSubcoreMesh(
        core_axis_name="c", subcore_axis_name="s",
        num_cores=_NC, num_subcores=_NS)


def _sc_degree(dst3):
    ones_rows = jnp.ones((_K, _D), jnp.float32)
    zeros_tile = jnp.zeros((_RPT, _D), jnp.float32)

    @functools.partial(
        pl.kernel,
        out_type=jax.ShapeDtypeStruct((_NC, _NPAD, _D), jnp.float32),
        mesh=_sc_mesh(),
        scratch_types=[
            pltpu.VMEM((_NCHUNK, _K), jnp.int32),
            pltpu.VMEM((_K, _D), jnp.float32),
            pltpu.VMEM_SHARED((_NPAD, _D), jnp.float32),
            pltpu.SemaphoreType.DMA,
        ],
    )
    def deg_kernel(dst_hbm, ones_hbm, zeros_hbm, out_hbm, didx, ones_v, deg_sh, sem):
        c = lax.axis_index("c")
        s = lax.axis_index("s")
        tile = c * _NS + s
        pltpu.sync_copy(ones_hbm, ones_v)
        pltpu.sync_copy(dst_hbm.at[tile], didx)
        pltpu.sync_copy(zeros_hbm, deg_sh.at[pl.ds(s * _RPT, _RPT)])
        plsc.subcore_barrier()

        def fire(j, carry):
            pltpu.async_copy(ones_v, deg_sh.at[didx.at[j]], sem, add=True)
            return carry

        def drain(j, carry):
            pltpu.make_async_copy(ones_v, deg_sh.at[didx.at[j]], sem).wait()
            return carry

        lax.fori_loop(0, _NCHUNK, fire, 0)
        lax.fori_loop(0, _NCHUNK, drain, 0)
        plsc.subcore_barrier()
        pltpu.sync_copy(deg_sh.at[pl.ds(s * _RPT, _RPT)],
                        out_hbm.at[c, pl.ds(s * _RPT, _RPT)])

    return deg_kernel(dst3, ones_rows, zeros_tile)


def _sc_edge_scatter(sd4, g):
    zeros_tile = jnp.zeros((_RPT, _D), jnp.float32)

    @functools.partial(
        pl.kernel,
        out_type=jax.ShapeDtypeStruct((_NC, _NPAD, _D), jnp.float32),
        mesh=_sc_mesh(),
        scratch_types=[
            pltpu.VMEM((2, _K), jnp.int32),
            pltpu.VMEM((2, _K), jnp.int32),
            pltpu.VMEM((_K, _D), jnp.float32),
            pltpu.VMEM((_K, _D), jnp.float32),
            pltpu.VMEM_SHARED((_NPAD, _D), jnp.float32),
            pltpu.SemaphoreType.DMA,
            pltpu.SemaphoreType.DMA,
            pltpu.SemaphoreType.DMA,
            pltpu.SemaphoreType.DMA,
        ],
    )
    def edge_kernel(sd_hbm, g_hbm, zeros_hbm, out_hbm,
                    sd0, sd1, rows0, rows1, agg_sh, gsem0, gsem1, ssem0, ssem1):
        c = lax.axis_index("c")
        s = lax.axis_index("s")
        tile = c * _NS + s
        pltpu.sync_copy(zeros_hbm, agg_sh.at[pl.ds(s * _RPT, _RPT)])
        plsc.subcore_barrier()

        pltpu.sync_copy(sd_hbm.at[tile, 0], sd0)
        pltpu.async_copy(g_hbm.at[sd0.at[0]], rows0, gsem0)
        pltpu.sync_copy(sd_hbm.at[tile, 1], sd1)
        pltpu.async_copy(g_hbm.at[sd1.at[0]], rows1, gsem1)

        def body(i, carry):
            j0 = 2 * i
            pltpu.make_async_copy(g_hbm.at[sd0.at[0]], rows0, gsem0).wait()
            pltpu.async_copy(rows0, agg_sh.at[sd0.at[1]], ssem0, add=True)
            pltpu.make_async_copy(rows0, agg_sh.at[sd0.at[1]], ssem0).wait()

            @pl.when(j0 + 2 < _NCHUNK)
            def _n0():
                pltpu.sync_copy(sd_hbm.at[tile, j0 + 2], sd0)
                pltpu.async_copy(g_hbm.at[sd0.at[0]], rows0, gsem0)

            pltpu.make_async_copy(g_hbm.at[sd1.at[0]], rows1, gsem1).wait()
            pltpu.async_copy(rows1, agg_sh.at[sd1.at[1]], ssem1, add=True)
            pltpu.make_async_copy(rows1, agg_sh.at[sd1.at[1]], ssem1).wait()

            @pl.when(j0 + 3 < _NCHUNK)
            def _n1():
                pltpu.sync_copy(sd_hbm.at[tile, j0 + 3], sd1)
                pltpu.async_copy(g_hbm.at[sd1.at[0]], rows1, gsem1)

            return carry

        lax.fori_loop(0, _NCHUNK // 2, body, 0)
        plsc.subcore_barrier()
        pltpu.sync_copy(agg_sh.at[pl.ds(s * _RPT, _RPT)],
                        out_hbm.at[c, pl.ds(s * _RPT, _RPT)])

    return edge_kernel(sd4, g, zeros_tile)


def _dinv_block(d_ref):
    deg = d_ref[0, :, 0:1] + d_ref[1, :, 0:1] + 1.0
    return lax.rsqrt(deg)


def _tc_first(x, W1, degp):
    def body(x_ref, w_ref, d_ref, o_ref):
        dinv = _dinv_block(d_ref)
        o_ref[...] = jnp.dot(x_ref[...], w_ref[...],
                             preferred_element_type=jnp.float32) * dinv

    return pl.pallas_call(
        body,
        grid=(_NB,),
        in_specs=[
            pl.BlockSpec((_R, _D), lambda i: (i, 0)),
            pl.BlockSpec((_D, _D), lambda i: (0, 0)),
            pl.BlockSpec((_NC, _R, _D), lambda i: (0, i, 0)),
        ],
        out_specs=pl.BlockSpec((_R, _D), lambda i: (i, 0)),
        out_shape=jax.ShapeDtypeStruct((_N, _D), jnp.float32),
    )(x, W1, degp)


def _tc_mid(acc, g, degp, b, W2):
    def body(a_ref, g_ref, d_ref, b_ref, w_ref, o_ref):
        dinv = _dinv_block(d_ref)
        agg = dinv * (a_ref[0] + a_ref[1] + g_ref[...]) + b_ref[...]
        x2 = jnp.maximum(agg, 0.0)
        o_ref[...] = jnp.dot(x2, w_ref[...],
                             preferred_element_type=jnp.float32) * dinv

    return pl.pallas_call(
        body,
        grid=(_NB,),
        in_specs=[
            pl.BlockSpec((_NC, _R, _D), lambda i: (0, i, 0)),
            pl.BlockSpec((_R, _D), lambda i: (i, 0)),
            pl.BlockSpec((_NC, _R, _D), lambda i: (0, i, 0)),
            pl.BlockSpec((1, _D), lambda i: (0, 0)),
            pl.BlockSpec((_D, _D), lambda i: (0, 0)),
        ],
        out_specs=pl.BlockSpec((_R, _D), lambda i: (i, 0)),
        out_shape=jax.ShapeDtypeStruct((_N, _D), jnp.float32),
    )(acc, g, degp, b, W2)


def _tc_final(acc, g, degp, b, batch3, fc1_W, fc1_b, fc2_Wp, fc2_bp):
    def body(a_ref, g_ref, d_ref, b_ref, bt_ref, w1_ref, b1_ref, w2_ref, b2_ref,
             o_ref, sum_sc, cnt_sc):
        i = pl.program_id(0)

        @pl.when(i == 0)
        def _init():
            sum_sc[...] = jnp.zeros_like(sum_sc)
            cnt_sc[...] = jnp.zeros_like(cnt_sc)

        dinv = _dinv_block(d_ref)
        agg = dinv * (a_ref[0] + a_ref[1] + g_ref[...]) + b_ref[...]
        x3 = jnp.maximum(agg, 0.0)
        bt = bt_ref[0, 0, :]
        onehot = (bt[None, :] ==
                  lax.broadcasted_iota(jnp.int32, (_G, _R), 0)).astype(jnp.float32)
        sum_sc[...] += jnp.dot(onehot, x3, preferred_element_type=jnp.float32)
        cnt_sc[...] += jnp.sum(onehot, axis=1, keepdims=True)

        @pl.when(i == _NB - 1)
        def _finish():
            pooled = sum_sc[...] / jnp.maximum(cnt_sc[...], 1.0)
            h = jnp.maximum(
                jnp.dot(pooled, w1_ref[...],
                        preferred_element_type=jnp.float32) + b1_ref[...], 0.0)
            logits = jnp.dot(h, w2_ref[...],
                             preferred_element_type=jnp.float32) + b2_ref[...]
            col = lax.broadcasted_iota(jnp.int32, (_G, _D), 1)
            logits = jnp.where(col < _OUT, logits, -1e30)
            m = jnp.max(logits, axis=1, keepdims=True)
            e = jnp.exp(logits - m)
            e = jnp.where(col < _OUT, e, 0.0)
            o_ref[...] = e / jnp.sum(e, axis=1, keepdims=True)

    return pl.pallas_call(
        body,
        grid=(_NB,),
        in_specs=[
            pl.BlockSpec((_NC, _R, _D), lambda i: (0, i, 0)),
            pl.BlockSpec((_R, _D), lambda i: (i, 0)),
            pl.BlockSpec((_NC, _R, _D), lambda i: (0, i, 0)),
            pl.BlockSpec((1, _D), lambda i: (0, 0)),
            pl.BlockSpec((1, 1, _R), lambda i: (i, 0, 0)),
            pl.BlockSpec((_D, _D), lambda i: (0, 0)),
            pl.BlockSpec((1, _D), lambda i: (0, 0)),
            pl.BlockSpec((_D, _D), lambda i: (0, 0)),
            pl.BlockSpec((1, _D), lambda i: (0, 0)),
        ],
        out_specs=pl.BlockSpec((_G, _D), lambda i: (0, 0)),
        out_shape=jax.ShapeDtypeStruct((_G, _D), jnp.float32),
        scratch_shapes=[
            pltpu.VMEM((_G, _D), jnp.float32),
            pltpu.VMEM((_G, 1), jnp.float32),
        ],
    )(acc, g, degp, b, batch3, fc1_W, fc1_b, fc2_Wp, fc2_bp)


def kernel(x, edge_index, batch, W1, b1, W2, b2, fc1_W, fc1_b, fc2_W, fc2_b):
    src = edge_index[0].astype(jnp.int32)
    dst = edge_index[1].astype(jnp.int32)
    batch3 = batch.astype(jnp.int32).reshape(_NB, 1, _R)

    pad = _EPAD - _E
    pad_idx = jnp.arange(pad, dtype=jnp.int32)
    src_p = jnp.concatenate([src, pad_idx % _N])
    dst_p = jnp.concatenate([dst, _N + pad_idx % (_NPAD - _N)])
    T = _NC * _NS
    sd4 = jnp.concatenate(
        [src_p.reshape(T, _NCHUNK, 1, _K), dst_p.reshape(T, _NCHUNK, 1, _K)],
        axis=2)
    dst3 = dst_p.reshape(T, _NCHUNK, _K)

    degp = _sc_degree(dst3)
    g1 = _tc_first(x, W1, degp)
    acc1 = _sc_edge_scatter(sd4, g1)
    g2 = _tc_mid(acc1, g1, degp, b1.reshape(1, _D), W2)
    acc2 = _sc_edge_scatter(sd4, g2)

    fc2_Wp = jnp.zeros((_D, _D), jnp.float32).at[:, :_OUT].set(fc2_W)
    fc2_bp = jnp.zeros((1, _D), jnp.float32).at[0, :_OUT].set(fc2_b)
    out = _tc_final(acc2, g2, degp, b2.reshape(1, _D), batch3,
                    fc1_W, fc1_b.reshape(1, _D), fc2_Wp, fc2_bp)
    return out[:, :_OUT]
